# Initial kernel scaffold; baseline (speedup 1.0000x reference)
import jax
import jax.numpy as jnp
from jax.experimental import pallas as pl


def kernel(x, table):
    raise NotImplementedError("write your pallas kernel here")


# speedup vs baseline: 4.6938x; 4.6938x over previous
import functools

import jax
import jax.numpy as jnp
from jax import lax
from jax.experimental import pallas as pl
from jax.experimental.pallas import tpu as pltpu
from jax.experimental.pallas import tpu_sc as plsc

_CHUNK = 128
_DPAD = 128


@functools.lru_cache(maxsize=None)
def _build_gather(V, D, B):
    info = plsc.get_sparse_core_info()
    NC, NS = info.num_cores, info.num_subcores
    NW = NC * NS
    assert B % (NW * _CHUNK) == 0, (B, NW)
    b_per_w = B // NW
    n_chunks = b_per_w // _CHUNK
    mesh = plsc.VectorSubcoreMesh(core_axis_name="c", subcore_axis_name="s")

    @functools.partial(
        pl.kernel,
        mesh=mesh,
        out_type=jax.ShapeDtypeStruct((B, _DPAD), jnp.float32),
        scratch_types=[
            pltpu.VMEM((b_per_w,), jnp.int32),
            pltpu.VMEM((_CHUNK, _DPAD), jnp.float32),
            pltpu.SemaphoreType.DMA,
        ],
    )
    def gather_kernel(table_hbm, idx_hbm, out_hbm, idx_v, rows_v, sem):
        wid = lax.axis_index("s") * NC + lax.axis_index("c")
        base = wid * b_per_w
        pltpu.sync_copy(idx_hbm.at[pl.ds(base, b_per_w)], idx_v)

        def body(c, carry):
            pltpu.async_copy(
                table_hbm.at[idx_v.at[pl.ds(c * _CHUNK, _CHUNK)]],
                rows_v, sem).wait()
            pltpu.sync_copy(rows_v,
                            out_hbm.at[pl.ds(base + c * _CHUNK, _CHUNK)])
            return carry

        lax.fori_loop(0, n_chunks, body, 0)

    return gather_kernel


def kernel(x, table):
    V, D = table.shape
    B = x.size
    idx = x.reshape(B).astype(jnp.int32)
    table_pad = jnp.pad(table, ((0, 0), (0, _DPAD - D)))
    out = _build_gather(V, D, B)(table_pad, idx)
    return out[:, :D].reshape(x.shape + (D,))


# speedup vs baseline: 6.0369x; 1.2861x over previous
import functools

import jax
import jax.numpy as jnp
from jax import lax
from jax.experimental import pallas as pl
from jax.experimental.pallas import tpu as pltpu
from jax.experimental.pallas import tpu_sc as plsc

_CHUNK = 128
_DPAD = 128
_NHALF = 2


@functools.lru_cache(maxsize=None)
def _build_gather(V, D, B):
    info = plsc.get_sparse_core_info()
    NC, NS = info.num_cores, info.num_subcores
    NW = NC * NS
    assert B % (NW * _NHALF * _CHUNK) == 0, (B, NW)
    b_per_w = B // NW
    b_half = b_per_w // _NHALF
    n_chunks = b_half // _CHUNK
    mesh = plsc.VectorSubcoreMesh(core_axis_name="c", subcore_axis_name="s")

    @functools.partial(
        pl.kernel,
        mesh=mesh,
        out_type=jax.ShapeDtypeStruct((B, _DPAD), jnp.float32),
        scratch_types=[
            pltpu.VMEM_SHARED((V, _DPAD), jnp.float32),
            pltpu.VMEM((b_half,), jnp.int32),
            pltpu.VMEM((_CHUNK, _DPAD), jnp.float32),
            pltpu.SemaphoreType.DMA,
        ],
    )
    def gather_kernel(table_hbm, idx_hbm, out_hbm, tab_s, idx_v, rows_v, sem):
        sid = lax.axis_index("s")
        wid = sid * NC + lax.axis_index("c")
        base = wid * b_per_w

        @pl.when(sid == 0)
        def _():
            pltpu.sync_copy(table_hbm, tab_s)

        plsc.subcore_barrier()

        for h in range(_NHALF):
            hbase = base + h * b_half
            pltpu.sync_copy(idx_hbm.at[pl.ds(hbase, b_half)], idx_v)

            def body(c, carry, hbase=hbase):
                pltpu.async_copy(
                    tab_s.at[idx_v.at[pl.ds(c * _CHUNK, _CHUNK)]],
                    rows_v, sem).wait()
                pltpu.sync_copy(
                    rows_v, out_hbm.at[pl.ds(hbase + c * _CHUNK, _CHUNK)])
                return carry

            lax.fori_loop(0, n_chunks, body, 0)

    return gather_kernel


def kernel(x, table):
    V, D = table.shape
    B = x.size
    idx = x.reshape(B).astype(jnp.int32)
    table_pad = jnp.pad(table, ((0, 0), (0, _DPAD - D)))
    out = _build_gather(V, D, B)(table_pad, idx)
    return out[:, :D].reshape(x.shape + (D,))


# speedup vs baseline: 7.3013x; 1.2094x over previous
import functools

import jax
import jax.numpy as jnp
from jax import lax
from jax.experimental import pallas as pl
from jax.experimental.pallas import tpu as pltpu
from jax.experimental.pallas import tpu_sc as plsc

_CHUNK = 128
_DPAD = 128
_NHALF = 2


@functools.lru_cache(maxsize=None)
def _build_gather(V, D, B):
    info = plsc.get_sparse_core_info()
    NC, NS = info.num_cores, info.num_subcores
    NW = NC * NS
    assert B % (NW * _NHALF * _CHUNK) == 0, (B, NW)
    b_per_w = B // NW
    b_half = b_per_w // _NHALF
    n_chunks = b_half // _CHUNK
    mesh = plsc.VectorSubcoreMesh(core_axis_name="c", subcore_axis_name="s")

    @functools.partial(
        pl.kernel,
        mesh=mesh,
        out_type=jax.ShapeDtypeStruct((B, _DPAD), jnp.float32),
        scratch_types=[
            pltpu.VMEM_SHARED((V, _DPAD), jnp.float32),
            pltpu.VMEM((b_half,), jnp.int32),
            pltpu.VMEM((_CHUNK, _DPAD), jnp.float32),
            pltpu.VMEM((_CHUNK, _DPAD), jnp.float32),
            pltpu.SemaphoreType.DMA,
            pltpu.SemaphoreType.DMA,
            pltpu.SemaphoreType.DMA,
        ],
    )
    def gather_kernel(table_hbm, idx_hbm, out_hbm, tab_s, idx_v,
                      rows0, rows1, sg, so0, so1):
        sid = lax.axis_index("s")
        wid = sid * NC + lax.axis_index("c")
        base = wid * b_per_w

        @pl.when(sid == 0)
        def _():
            pltpu.sync_copy(table_hbm, tab_s)

        plsc.subcore_barrier()

        bufs = (rows0, rows1)
        osems = (so0, so1)

        def gather(c, b):
            pltpu.async_copy(
                tab_s.at[idx_v.at[pl.ds(c * _CHUNK, _CHUNK)]],
                bufs[b], sg).wait()

        def start_out(hbase, c, b):
            pltpu.async_copy(
                bufs[b], out_hbm.at[pl.ds(hbase + c * _CHUNK, _CHUNK)],
                osems[b])

        def wait_out(b):
            pltpu.make_async_copy(
                bufs[b], out_hbm.at[pl.ds(base, _CHUNK)], osems[b]).wait()

        for h in range(_NHALF):
            hbase = base + h * b_half
            pltpu.sync_copy(idx_hbm.at[pl.ds(hbase, b_half)], idx_v)

            gather(0, 0)
            start_out(hbase, 0, 0)
            gather(1, 1)
            start_out(hbase, 1, 1)

            def body(p, carry, hbase=hbase):
                for b in range(2):
                    c = 2 * p + b
                    wait_out(b)
                    gather(c, b)
                    start_out(hbase, c, b)
                return carry

            lax.fori_loop(1, n_chunks // 2, body, 0)
            wait_out(0)
            wait_out(1)

    return gather_kernel


def kernel(x, table):
    V, D = table.shape
    B = x.size
    idx = x.reshape(B).astype(jnp.int32)
    table_pad = jnp.pad(table, ((0, 0), (0, _DPAD - D)))
    out = _build_gather(V, D, B)(table_pad, idx)
    return out[:, :D].reshape(x.shape + (D,))
